# Initial kernel scaffold; baseline (speedup 1.0000x reference)
import jax
import jax.numpy as jnp
from jax.experimental import pallas as pl


def kernel(mention_vectors, concept_indices, concept_lengths, W1, b1, W2, b2):
    raise NotImplementedError("write your pallas kernel here")


# speedup vs baseline: 2.4444x; 2.4444x over previous
import functools

import jax
import jax.numpy as jnp
from jax import lax
from jax.experimental import pallas as pl
from jax.experimental.pallas import tpu as pltpu
from jax.experimental.pallas import tpu_sc as plsc

_NC = 2
_NS = 16
_NW = _NC * _NS
_LANES = 16


def _ff_body(x_ref, w1_ref, b1_ref, w2_ref, o_ref):
    x = x_ref[...]
    h = jnp.dot(x, w1_ref[...], preferred_element_type=jnp.float32)
    h = jnp.maximum(h + b1_ref[...][None, :], 0.0)
    s = jnp.dot(h, w2_ref[...], preferred_element_type=jnp.float32)
    o_ref[...] = s[:, 0]


def _ff_scores(mv, W1, b1, W2):
    n, d = mv.shape
    hdim = W1.shape[1]
    blk = 2048
    grid = n // blk
    return pl.pallas_call(
        _ff_body,
        grid=(grid,),
        in_specs=[
            pl.BlockSpec((blk, d), lambda i: (i, 0)),
            pl.BlockSpec((d, hdim), lambda i: (0, 0)),
            pl.BlockSpec((hdim,), lambda i: (0,)),
            pl.BlockSpec((hdim, 1), lambda i: (0, 0)),
        ],
        out_specs=pl.BlockSpec((blk,), lambda i: (i,)),
        out_shape=jax.ShapeDtypeStruct((n,), jnp.float32),
    )(mv, W1, b1, W2)


def _sc_attention(scores, idx, lengths, mv):
    C, L = idx.shape
    N, D = mv.shape
    cpt = C // _NW
    nchunk = D // _LANES

    mesh = plsc.VectorSubcoreMesh(core_axis_name="c", subcore_axis_name="s")

    @functools.partial(
        pl.kernel,
        mesh=mesh,
        compiler_params=pltpu.CompilerParams(needs_layout_passes=False),
        out_type=jax.ShapeDtypeStruct((C, D), jnp.float32),
        scratch_types=[
            pltpu.VMEM((N,), jnp.float32),
            pltpu.VMEM((cpt, L), jnp.int32),
            pltpu.VMEM((cpt + _LANES,), jnp.int32),
            pltpu.VMEM((L, D), jnp.float32),
            pltpu.VMEM((L + _LANES,), jnp.float32),
            pltpu.VMEM((D,), jnp.float32),
            pltpu.SemaphoreType.DMA,
        ],
    )
    def k(scores_hbm, idx_hbm, len_hbm, mv_hbm, out_hbm,
          scores_v, idx_v, len_v, rows_v, probs_v, acc_v, gsem):
        wid = lax.axis_index("s") * _NC + lax.axis_index("c")
        base = wid * cpt
        pltpu.sync_copy(scores_hbm, scores_v)
        pltpu.sync_copy(idx_hbm.at[pl.ds(base, cpt)], idx_v)
        pltpu.sync_copy(len_hbm.at[pl.ds(base, cpt)], len_v.at[pl.ds(0, cpt)])

        lane = lax.iota(jnp.int32, _LANES)

        def per_concept(c, carry):
            cp = pltpu.async_copy(mv_hbm.at[idx_v.at[c]], rows_v, gsem)

            i0 = idx_v[c, pl.ds(0, _LANES)]
            i1 = idx_v[c, pl.ds(_LANES, _LANES)]
            s0 = plsc.load_gather(scores_v, [i0])
            s1 = plsc.load_gather(scores_v, [i1])
            ln = len_v[pl.ds(c, _LANES)][0]
            lnv = jnp.full((_LANES,), ln, dtype=jnp.int32)
            m0 = lane < lnv
            m1 = (lane + _LANES) < lnv
            s0 = jnp.where(m0, s0, s0 - 1e38)
            s1 = jnp.where(m1, s1, s1 - 1e38)
            mx = jnp.max(jnp.maximum(s0, s1))
            e0 = jnp.exp(s0 - mx)
            e1 = jnp.exp(s1 - mx)
            denom = jnp.sum(e0 + e1)
            probs_v[pl.ds(0, _LANES)] = e0 / denom
            probs_v[pl.ds(_LANES, _LANES)] = e1 / denom

            cp.wait()

            def fma(l, accs):
                pvec = jnp.full((_LANES,), probs_v[pl.ds(l, _LANES)][0],
                                dtype=jnp.float32)
                return tuple(
                    accs[j] + pvec * rows_v[l, pl.ds(j * _LANES, _LANES)]
                    for j in range(nchunk)
                )

            zero = jnp.zeros((_LANES,), jnp.float32)
            accs = lax.fori_loop(0, L, fma, tuple(zero for _ in range(nchunk)))
            for j in range(nchunk):
                acc_v[pl.ds(j * _LANES, _LANES)] = accs[j]
            pltpu.sync_copy(acc_v, out_hbm.at[base + c])
            return carry

        lax.fori_loop(0, cpt, per_concept, 0)

    return k(scores, idx, lengths, mv)


def kernel(mention_vectors, concept_indices, concept_lengths, W1, b1, W2, b2):
    num_batch, m, d = mention_vectors.shape
    mv = mention_vectors.reshape(-1, d)
    scores = _ff_scores(mv, W1, b1, W2)
    out = _sc_attention(scores, concept_indices, concept_lengths, mv)
    return out.reshape(num_batch, -1, d)


# speedup vs baseline: 4.8574x; 1.9872x over previous
import functools

import jax
import jax.numpy as jnp
from jax import lax
from jax.experimental import pallas as pl
from jax.experimental.pallas import tpu as pltpu
from jax.experimental.pallas import tpu_sc as plsc

_NC = 2
_NS = 16
_NW = _NC * _NS
_LANES = 16


def _ff_body(x_ref, w1_ref, b1_ref, w2_ref, o_ref):
    x = x_ref[...]
    h = jnp.dot(x, w1_ref[...], preferred_element_type=jnp.float32)
    h = jnp.maximum(h + b1_ref[...][None, :], 0.0)
    s = jnp.dot(h, w2_ref[...], preferred_element_type=jnp.float32)
    o_ref[...] = s[:, 0]


def _ff_scores(mv, W1, b1, W2):
    n, d = mv.shape
    hdim = W1.shape[1]
    blk = 2048
    grid = n // blk
    return pl.pallas_call(
        _ff_body,
        grid=(grid,),
        in_specs=[
            pl.BlockSpec((blk, d), lambda i: (i, 0)),
            pl.BlockSpec((d, hdim), lambda i: (0, 0)),
            pl.BlockSpec((hdim,), lambda i: (0,)),
            pl.BlockSpec((hdim, 1), lambda i: (0, 0)),
        ],
        out_specs=pl.BlockSpec((blk,), lambda i: (i,)),
        out_shape=jax.ShapeDtypeStruct((n,), jnp.float32),
    )(mv, W1, b1, W2)


def _sc_attention(scores, idx, lengths, mv):
    C, L = idx.shape
    N, D = mv.shape
    cpt = C // _NW
    nchunk = D // _LANES

    mesh = plsc.VectorSubcoreMesh(core_axis_name="c", subcore_axis_name="s")

    grp = 8

    @functools.partial(
        pl.kernel,
        mesh=mesh,
        compiler_params=pltpu.CompilerParams(needs_layout_passes=False),
        out_type=jax.ShapeDtypeStruct((C, D), jnp.float32),
        scratch_types=[
            pltpu.VMEM((N,), jnp.float32),
            pltpu.VMEM((cpt, L), jnp.int32),
            pltpu.VMEM((cpt + _LANES,), jnp.int32),
            pltpu.VMEM((L, D), jnp.float32),
            pltpu.VMEM((L, D), jnp.float32),
            pltpu.VMEM((L + _LANES,), jnp.float32),
            pltpu.VMEM((D,), jnp.float32),
            pltpu.VMEM((D,), jnp.float32),
            pltpu.SemaphoreType.DMA,
            pltpu.SemaphoreType.DMA,
            pltpu.SemaphoreType.DMA,
            pltpu.SemaphoreType.DMA,
        ],
    )
    def k(scores_hbm, idx_hbm, len_hbm, mv_hbm, out_hbm,
          scores_v, idx_v, len_v, rows0_v, rows1_v, probs_v, acc0_v, acc1_v,
          gsem0, gsem1, osem0, osem1):
        wid = lax.axis_index("s") * _NC + lax.axis_index("c")
        base = wid * cpt
        pltpu.sync_copy(scores_hbm, scores_v)
        pltpu.sync_copy(idx_hbm.at[pl.ds(base, cpt)], idx_v)
        pltpu.sync_copy(len_hbm.at[pl.ds(base, cpt)], len_v.at[pl.ds(0, cpt)])

        lane = lax.iota(jnp.int32, _LANES)

        def nchunks(c):
            ln = len_v[pl.ds(c, _LANES)][0]
            return (ln + (grp - 1)) // grp

        def fire(c, rows_ref, sem):
            def body(i, carry):
                pltpu.async_copy(
                    mv_hbm.at[idx_v.at[c, pl.ds(i * grp, grp)]],
                    rows_ref.at[pl.ds(i * grp, grp)], sem)
                return carry
            lax.fori_loop(0, nchunks(c), body, 0)

        def wait_rows(c, rows_ref, sem):
            def body(i, carry):
                pltpu.make_async_copy(
                    mv_hbm.at[idx_v.at[c, pl.ds(0, grp)]],
                    rows_ref.at[pl.ds(0, grp)], sem).wait()
                return carry
            lax.fori_loop(0, nchunks(c), body, 0)

        def process(c, rows_ref, gsem, acc_ref, osem, wait_out):
            i0 = idx_v[c, pl.ds(0, _LANES)]
            i1 = idx_v[c, pl.ds(_LANES, _LANES)]
            s0 = plsc.load_gather(scores_v, [i0])
            s1 = plsc.load_gather(scores_v, [i1])
            ln = len_v[pl.ds(c, _LANES)][0]
            lnv = jnp.full((_LANES,), ln, dtype=jnp.int32)
            m0 = lane < lnv
            m1 = (lane + _LANES) < lnv
            s0 = jnp.where(m0, s0, s0 - 1e38)
            s1 = jnp.where(m1, s1, s1 - 1e38)
            mx = jnp.max(jnp.maximum(s0, s1))
            e0 = jnp.exp(s0 - mx)
            e1 = jnp.exp(s1 - mx)
            denom = jnp.sum(e0 + e1)
            probs_v[pl.ds(0, _LANES)] = e0 / denom
            probs_v[pl.ds(_LANES, _LANES)] = e1 / denom

            wait_rows(c, rows_ref, gsem)

            def fma(l, accs):
                pvec = jnp.full((_LANES,), probs_v[pl.ds(l, _LANES)][0],
                                dtype=jnp.float32)
                return tuple(
                    accs[j] + pvec * rows_ref[l, pl.ds(j * _LANES, _LANES)]
                    for j in range(nchunk)
                )

            zero = jnp.zeros((_LANES,), jnp.float32)
            accs = lax.fori_loop(0, ln, fma, tuple(zero for _ in range(nchunk)))

            @pl.when(wait_out)
            def _():
                pltpu.make_async_copy(acc_ref, out_hbm.at[base], osem).wait()
            for j in range(nchunk):
                acc_ref[pl.ds(j * _LANES, _LANES)] = accs[j]
            pltpu.async_copy(acc_ref, out_hbm.at[base + c], osem)

        npairs = cpt // 2
        fire(0, rows0_v, gsem0)

        def pair(g, carry):
            c0 = 2 * g
            fire(c0 + 1, rows1_v, gsem1)
            process(c0, rows0_v, gsem0, acc0_v, osem0, g > 0)
            @pl.when(g + 1 < npairs)
            def _():
                fire(c0 + 2, rows0_v, gsem0)
            process(c0 + 1, rows1_v, gsem1, acc1_v, osem1, g > 0)
            return carry

        lax.fori_loop(0, npairs, pair, 0)
        pltpu.make_async_copy(acc0_v, out_hbm.at[base], osem0).wait()
        pltpu.make_async_copy(acc1_v, out_hbm.at[base], osem1).wait()

    return k(scores, idx, lengths, mv)


def kernel(mention_vectors, concept_indices, concept_lengths, W1, b1, W2, b2):
    num_batch, m, d = mention_vectors.shape
    mv = mention_vectors.reshape(-1, d)
    scores = _ff_scores(mv, W1, b1, W2)
    out = _sc_attention(scores, concept_indices, concept_lengths, mv)
    return out.reshape(num_batch, -1, d)
